# Initial kernel scaffold; baseline (speedup 1.0000x reference)
import jax
import jax.numpy as jnp
from jax.experimental import pallas as pl


def kernel(x, table):
    raise NotImplementedError("write your pallas kernel here")


# speedup vs baseline: 4.1512x; 4.1512x over previous
import functools

import jax
import jax.numpy as jnp
from jax import lax
from jax.experimental import pallas as pl
from jax.experimental.pallas import tpu as pltpu
from jax.experimental.pallas import tpu_sc as plsc

NUM_EMB = 100000
DIM = 64
B_TOTAL = 4096 * 50
NC = 2
NS = 16
NW = NC * NS
PER_W = B_TOTAL // NW
CHUNK = 128
NCH = PER_W // CHUNK
NBUF = 6
AHEAD = 3

_mesh = plsc.VectorSubcoreMesh(core_axis_name="c", subcore_axis_name="s")


@functools.partial(
    pl.kernel,
    mesh=_mesh,
    out_type=jax.ShapeDtypeStruct((B_TOTAL, DIM), jnp.float32),
    scratch_types=[
        pltpu.VMEM((NCH, CHUNK), jnp.int32),
        pltpu.VMEM((NBUF, CHUNK, DIM), jnp.float32),
    ]
    + [pltpu.SemaphoreType.DMA] * (2 * NBUF + 1),
    compiler_params=pltpu.CompilerParams(use_tc_tiling_on_sc=False),
)
def _emb_gather(x_hbm, table_hbm, out_hbm, idx_v, bufs, *sems):
    idx_sem = sems[0]
    gsems = sems[1 : 1 + NBUF]
    psems = sems[1 + NBUF :]
    wid = lax.axis_index("s") * NC + lax.axis_index("c")
    base = wid * PER_W

    pltpu.async_copy(x_hbm.at[wid], idx_v, idx_sem).wait()

    def _remap(r, carry):
        for c in range(CHUNK // 16):
            v = idx_v[r, pl.ds(c * 16, 16)]
            idx_v[r, pl.ds(c * 16, 16)] = jnp.where(v == 0, NUM_EMB, v)
        return carry

    lax.fori_loop(0, NCH, _remap, 0)

    hg = [None] * NCH
    hp = [None] * NCH
    for t in range(NCH + AHEAD):
        g = t
        if g < NCH:
            b = g % NBUF
            if g - NBUF >= 0:
                hp[g - NBUF].wait()
            hg[g] = pltpu.async_copy(
                table_hbm.at[idx_v.at[g]], bufs.at[b], gsems[b]
            )
        d = t - AHEAD
        if 0 <= d < NCH:
            b = d % NBUF
            hg[d].wait()
            hp[d] = pltpu.async_copy(
                bufs.at[b], out_hbm.at[pl.ds(base + d * CHUNK, CHUNK)], psems[b]
            )
    for d in range(NCH - NBUF, NCH):
        hp[d].wait()


def kernel(x, table):
    xf = x.reshape(NW, NCH, CHUNK).astype(jnp.int32)
    tablep = jnp.concatenate(
        [table, jnp.zeros((1, DIM), dtype=table.dtype)], axis=0
    )
    out = _emb_gather(xf, tablep)
    return out.reshape(x.shape[0], x.shape[1], DIM)


# speedup vs baseline: 4.6192x; 1.1127x over previous
import functools

import jax
import jax.numpy as jnp
from jax import lax
from jax.experimental import pallas as pl
from jax.experimental.pallas import tpu as pltpu
from jax.experimental.pallas import tpu_sc as plsc

DIM = 64
B_TOTAL = 4096 * 50
NC = 2
NS = 16
NW = NC * NS
PER_W = B_TOTAL // NW
CHUNK = 128
NCH = PER_W // CHUNK
NBUF = 6
AHEAD = 3

_mesh = plsc.VectorSubcoreMesh(core_axis_name="c", subcore_axis_name="s")


@functools.partial(
    pl.kernel,
    mesh=_mesh,
    out_type=jax.ShapeDtypeStruct((B_TOTAL, DIM), jnp.float32),
    scratch_types=[
        pltpu.VMEM((NCH, CHUNK), jnp.int32),
        pltpu.VMEM((NBUF, CHUNK, DIM), jnp.float32),
        pltpu.VMEM((16, DIM), jnp.float32),
        pltpu.VMEM((PER_W + 16,), jnp.int32),
    ]
    + [pltpu.SemaphoreType.DMA] * (2 * NBUF + 2),
    compiler_params=pltpu.CompilerParams(
        use_tc_tiling_on_sc=False, needs_layout_passes=False
    ),
)
def _emb_gather(x_hbm, table_hbm, out_hbm, idx_v, bufs, zrow, plist, *sems):
    idx_sem = sems[0]
    fix_sem = sems[1]
    gsems = sems[2 : 2 + NBUF]
    psems = sems[2 + NBUF :]
    wid = lax.axis_index("s") * NC + lax.axis_index("c")
    base = wid * PER_W

    pltpu.async_copy(x_hbm.at[wid], idx_v, idx_sem).wait()
    zeros16 = jnp.zeros((16,), jnp.float32)
    for r in range(16):
        for c in range(DIM // 16):
            zrow[r, pl.ds(c * 16, 16)] = zeros16

    hg = [None] * NCH
    hp = [None] * NCH
    for t in range(NCH + AHEAD):
        g = t
        if g < NCH:
            b = g % NBUF
            if g - NBUF >= 0:
                hp[g - NBUF].wait()
            hg[g] = pltpu.async_copy(
                table_hbm.at[idx_v.at[g]], bufs.at[b], gsems[b]
            )
        d = t - AHEAD
        if 0 <= d < NCH:
            b = d % NBUF
            hg[d].wait()
            hp[d] = pltpu.async_copy(
                bufs.at[b], out_hbm.at[pl.ds(base + d * CHUNK, CHUNK)], psems[b]
            )
    for d in range(NCH - NBUF, NCH):
        hp[d].wait()

    lanes = lax.iota(jnp.int32, 16)
    big = jnp.int32(2**30)

    def _compact(i, carry):
        off, first = carry
        d = i // (CHUNK // 16)
        g = i % (CHUNK // 16)
        v = idx_v[d, pl.ds(g * 16, 16)]
        m = v == 0
        pos = base + i * 16 + lanes
        first = jnp.minimum(first, jnp.min(jnp.where(m, pos, big)))
        plsc.store_compressed(plist.at[pl.ds(off, 16)], pos, mask=m)
        cnt = plsc.all_reduce_population_count(m)[0]
        return off + cnt, first

    npad, first = lax.fori_loop(
        0, PER_W // 16, _compact, (jnp.int32(0), big)
    )
    plist[pl.ds(npad, 16)] = jnp.full((16,), first, jnp.int32)

    def _scatter_zeros(j, carry):
        tv = plist[pl.ds(j * 16, 16)]
        pltpu.async_copy(zrow, out_hbm.at[tv], fix_sem).wait()
        return carry

    lax.fori_loop(0, (npad + 15) // 16, _scatter_zeros, 0, unroll=False)


def kernel(x, table):
    xf = x.reshape(NW, NCH, CHUNK).astype(jnp.int32)
    out = _emb_gather(xf, table)
    return out.reshape(x.shape[0], x.shape[1], DIM)
